# Initial kernel scaffold; baseline (speedup 1.0000x reference)
import jax
import jax.numpy as jnp
from jax.experimental import pallas as pl


def kernel(x, edge_index, edge_type, W1, root1, b1, W2, root2, b2):
    raise NotImplementedError("write your pallas kernel here")


# speedup vs baseline: 10.0051x; 10.0051x over previous
import functools

import jax
import jax.numpy as jnp
from jax import lax
from jax.experimental import pallas as pl
from jax.experimental.pallas import tpu as pltpu
from jax.experimental.pallas import tpu_sc as plsc

N = 10000
E = 320000
D_IN = 128
HID = 64
D_OUT = 128
R = 8

NW = 32
E_PAD = 327680
EW = E_PAD // NW
C = 128
NCHUNK = EW // C
NKEY = N * R
NKEY_PAD = 81920
CNT_SLICE = NKEY_PAD // 16
ACC_ROWS = N + 16
ZROWS = 624
ZBUF_ROWS = 104

_MESH = plsc.VectorSubcoreMesh(core_axis_name="c", subcore_axis_name="s")
_SC_PARAMS = pltpu.CompilerParams(use_tc_tiling_on_sc=False)


def _counts_body(dst_hbm, typ_hbm, out_hbm, dstv, typv, keyv, ones, zbuf, acc):
    cid = lax.axis_index("c")
    sid = lax.axis_index("s")
    wid = sid * 2 + cid

    def init_ones(i, _):
        ones[pl.ds(i * 16, 16)] = jnp.ones((16,), jnp.float32)
        return 0

    lax.fori_loop(0, C // 16, init_ones, 0)

    def init_z(i, _):
        zbuf[pl.ds(i * 16, 16)] = jnp.zeros((16,), jnp.float32)
        return 0

    lax.fori_loop(0, CNT_SLICE // 16, init_z, 0)
    pltpu.sync_copy(zbuf, acc.at[pl.ds(sid * CNT_SLICE, CNT_SLICE)])
    plsc.subcore_barrier()

    def chunk(k, _):
        base = pl.multiple_of(wid * EW + k * C, C)
        pltpu.sync_copy(dst_hbm.at[pl.ds(base, C)], dstv)
        pltpu.sync_copy(typ_hbm.at[pl.ds(base, C)], typv)

        def ckey(i, _):
            sl = pl.ds(i * 16, 16)
            keyv[sl] = dstv[sl] * R + typv[sl]
            return 0

        lax.fori_loop(0, C // 16, ckey, 0)
        pltpu.sync_copy(ones, acc.at[keyv], add=True)
        return 0

    lax.fori_loop(0, NCHUNK, chunk, 0)
    plsc.subcore_barrier()
    pltpu.sync_copy(acc.at[pl.ds(sid * CNT_SLICE, CNT_SLICE)],
                    out_hbm.at[cid, 0, pl.ds(sid * CNT_SLICE, CNT_SLICE)])


_sc_counts = functools.partial(
    pl.kernel,
    out_type=jax.ShapeDtypeStruct((2, 1, NKEY_PAD), jnp.float32),
    mesh=_MESH,
    compiler_params=_SC_PARAMS,
    scratch_types=[
        pltpu.VMEM((C,), jnp.int32),
        pltpu.VMEM((C,), jnp.int32),
        pltpu.VMEM((C,), jnp.int32),
        pltpu.VMEM((C,), jnp.float32),
        pltpu.VMEM((CNT_SLICE,), jnp.float32),
        pltpu.VMEM_SHARED((NKEY_PAD,), jnp.float32),
    ],
)(_counts_body)


def _make_edge_body(H):
    def body(src_hbm, dst_hbm, typ_hbm, tab_hbm, inv_hbm, out_hbm,
             srcv, dstv, typv, rowi, keyv, wv, rows, zbuf, acc, sem1, sem2):
        cid = lax.axis_index("c")
        sid = lax.axis_index("s")
        wid = sid * 2 + cid

        def zrow(r, _):
            for c in range(H // 16):
                zbuf[r, pl.ds(c * 16, 16)] = jnp.zeros((16,), jnp.float32)
            return 0

        lax.fori_loop(0, ZBUF_ROWS, zrow, 0)
        for z in range(ZROWS // ZBUF_ROWS):
            pltpu.sync_copy(zbuf,
                            acc.at[pl.ds(sid * ZROWS + z * ZBUF_ROWS,
                                         ZBUF_ROWS)])

        @pl.when(sid == 0)
        def _zero_tail():
            pltpu.sync_copy(zbuf.at[pl.ds(0, ACC_ROWS - 16 * ZROWS)],
                            acc.at[pl.ds(16 * ZROWS, ACC_ROWS - 16 * ZROWS)])

        plsc.subcore_barrier()

        def chunk(k, _):
            base = pl.multiple_of(wid * EW + k * C, C)
            pltpu.sync_copy(src_hbm.at[pl.ds(base, C)], srcv)
            pltpu.sync_copy(dst_hbm.at[pl.ds(base, C)], dstv)
            pltpu.sync_copy(typ_hbm.at[pl.ds(base, C)], typv)

            def ckey(i, _):
                sl = pl.ds(i * 16, 16)
                t = typv[sl]
                rowi[sl] = srcv[sl] * R + t
                keyv[sl] = dstv[sl] * R + t
                return 0

            lax.fori_loop(0, C // 16, ckey, 0)
            g1 = pltpu.async_copy(tab_hbm.at[rowi], rows, sem1)
            g2 = pltpu.async_copy(inv_hbm.at[keyv], wv, sem2)
            g1.wait()
            g2.wait()

            def scale(e, _):
                w = wv[e]
                for c in range(H // 16):
                    sl = pl.ds(c * 16, 16)
                    rows[e, sl] = rows[e, sl] * w
                return 0

            lax.fori_loop(0, C, scale, 0)
            pltpu.sync_copy(rows, acc.at[dstv], add=True)
            return 0

        lax.fori_loop(0, NCHUNK, chunk, 0)
        plsc.subcore_barrier()
        pltpu.sync_copy(acc.at[pl.ds(sid * ZROWS, ZROWS)],
                        out_hbm.at[cid, pl.ds(sid * ZROWS, ZROWS)])

        @pl.when(sid == 0)
        def _copy_tail():
            pltpu.sync_copy(acc.at[pl.ds(16 * ZROWS, N - 16 * ZROWS)],
                            out_hbm.at[cid, pl.ds(16 * ZROWS, N - 16 * ZROWS)])

    return body


def _sc_edge(H):
    return functools.partial(
        pl.kernel,
        out_type=jax.ShapeDtypeStruct((2, N, H), jnp.float32),
        mesh=_MESH,
        compiler_params=_SC_PARAMS,
        scratch_types=[
            pltpu.VMEM((C,), jnp.int32),
            pltpu.VMEM((C,), jnp.int32),
            pltpu.VMEM((C,), jnp.int32),
            pltpu.VMEM((C,), jnp.int32),
            pltpu.VMEM((C,), jnp.int32),
            pltpu.VMEM((C, 16), jnp.float32),
            pltpu.VMEM((C, H), jnp.float32),
            pltpu.VMEM((ZBUF_ROWS, H), jnp.float32),
            pltpu.VMEM_SHARED((ACC_ROWS, H), jnp.float32),
            pltpu.SemaphoreType.DMA,
            pltpu.SemaphoreType.DMA,
        ],
    )(_make_edge_body(H))


_sc_edge64 = _sc_edge(HID)
_sc_edge128 = _sc_edge(D_OUT)


def _inv_body(cnt_ref, inv_ref):
    c = cnt_ref[0] + cnt_ref[1]
    inv_ref[...] = jnp.where(c > 0.0, 1.0 / jnp.maximum(c, 1.0), 0.0)


def _tc_inv(cnt):
    return pl.pallas_call(
        _inv_body,
        out_shape=jax.ShapeDtypeStruct((NKEY_PAD,), jnp.float32),
    )(cnt)


def _mm1_body(x_ref, w_ref, b_ref, p_ref, t_ref):
    acc = jnp.dot(x_ref[...], w_ref[...],
                  preferred_element_type=jnp.float32) + b_ref[...]
    p_ref[...] = acc[:, :HID]
    t_ref[...] = acc[:, HID:]


def _tc_mm1(x, wcat, bcat):
    blk = 1000
    return pl.pallas_call(
        _mm1_body,
        grid=(N // blk,),
        in_specs=[
            pl.BlockSpec((blk, D_IN), lambda i: (i, 0)),
            pl.BlockSpec((D_IN, HID + R * HID), lambda i: (0, 0)),
            pl.BlockSpec((1, HID + R * HID), lambda i: (0, 0)),
        ],
        out_specs=[
            pl.BlockSpec((blk, HID), lambda i: (i, 0)),
            pl.BlockSpec((blk, R * HID), lambda i: (i, 0)),
        ],
        out_shape=[
            jax.ShapeDtypeStruct((N, HID), jnp.float32),
            jax.ShapeDtypeStruct((N, R * HID), jnp.float32),
        ],
    )(x, wcat, bcat)


def _mm2_body(p1_ref, a0_ref, a1_ref, w_ref, b_ref, p_ref, t_ref):
    h = jnp.maximum(p1_ref[...] + a0_ref[...] + a1_ref[...], 0.0)
    acc = jnp.dot(h, w_ref[...],
                  preferred_element_type=jnp.float32) + b_ref[...]
    p_ref[...] = acc[:, :D_OUT]
    t_ref[...] = acc[:, D_OUT:]


def _tc_mm2(p1, a0, a1, wcat, bcat):
    blk = 1000
    return pl.pallas_call(
        _mm2_body,
        grid=(N // blk,),
        in_specs=[
            pl.BlockSpec((blk, HID), lambda i: (i, 0)),
            pl.BlockSpec((blk, HID), lambda i: (i, 0)),
            pl.BlockSpec((blk, HID), lambda i: (i, 0)),
            pl.BlockSpec((HID, D_OUT + R * D_OUT), lambda i: (0, 0)),
            pl.BlockSpec((1, D_OUT + R * D_OUT), lambda i: (0, 0)),
        ],
        out_specs=[
            pl.BlockSpec((blk, D_OUT), lambda i: (i, 0)),
            pl.BlockSpec((blk, R * D_OUT), lambda i: (i, 0)),
        ],
        out_shape=[
            jax.ShapeDtypeStruct((N, D_OUT), jnp.float32),
            jax.ShapeDtypeStruct((N, R * D_OUT), jnp.float32),
        ],
    )(p1, a0, a1, wcat, bcat)


def _add3_body(p_ref, a0_ref, a1_ref, o_ref):
    o_ref[...] = p_ref[...] + a0_ref[...] + a1_ref[...]


def _tc_add3(p, a0, a1):
    blk = 1000
    return pl.pallas_call(
        _add3_body,
        grid=(N // blk,),
        in_specs=[pl.BlockSpec((blk, D_OUT), lambda i: (i, 0))] * 3,
        out_specs=pl.BlockSpec((blk, D_OUT), lambda i: (i, 0)),
        out_shape=jax.ShapeDtypeStruct((N, D_OUT), jnp.float32),
    )(p, a0, a1)


def kernel(x, edge_index, edge_type, W1, root1, b1, W2, root2, b2):
    src = edge_index[0].astype(jnp.int32)
    dst = edge_index[1].astype(jnp.int32)
    typ = edge_type.astype(jnp.int32)
    pad = E_PAD - E
    src_p = jnp.concatenate([src, jnp.zeros((pad,), jnp.int32)])
    dst_p = jnp.concatenate([dst, jnp.full((pad,), N, jnp.int32)])
    typ_p = jnp.concatenate([typ, jnp.zeros((pad,), jnp.int32)])

    wcat1 = jnp.concatenate(
        [root1, jnp.transpose(W1, (1, 0, 2)).reshape(D_IN, R * HID)], axis=1)
    bcat1 = jnp.concatenate(
        [b1, jnp.zeros((R * HID,), jnp.float32)])[None, :]
    wcat2 = jnp.concatenate(
        [root2, jnp.transpose(W2, (1, 0, 2)).reshape(HID, R * D_OUT)], axis=1)
    bcat2 = jnp.concatenate(
        [b2, jnp.zeros((R * D_OUT,), jnp.float32)])[None, :]

    cnt = _sc_counts(dst_p, typ_p).reshape(2, NKEY_PAD)
    inv = _tc_inv(cnt)
    inv16 = jnp.broadcast_to(inv[:, None], (NKEY_PAD, 16))

    p1, t1 = _tc_mm1(x, wcat1, bcat1)
    a1 = _sc_edge64(src_p, dst_p, typ_p, t1.reshape(N * R, HID), inv16)
    p2, t2 = _tc_mm2(p1, a1[0], a1[1], wcat2, bcat2)
    a2 = _sc_edge128(src_p, dst_p, typ_p, t2.reshape(N * R, D_OUT), inv16)
    return _tc_add3(p2, a2[0], a2[1])


# speedup vs baseline: 13.8987x; 1.3892x over previous
import functools

import jax
import jax.numpy as jnp
from jax import lax
from jax.experimental import pallas as pl
from jax.experimental.pallas import tpu as pltpu
from jax.experimental.pallas import tpu_sc as plsc

N = 10000
E = 320000
D_IN = 128
HID = 64
D_OUT = 128
R = 8

NW = 32
E_PAD = 327680
EW = E_PAD // NW
C = 128
NCHUNK = EW // C
NSUPER = 8
SUPER = NCHUNK // NSUPER
NKEY = N * R
NKEY_PAD = 81920
CNT_SLICE = NKEY_PAD // 16
ACC_ROWS = N + 16
ZROWS = 624
ZBUF_ROWS = 8

_MESH = plsc.VectorSubcoreMesh(core_axis_name="c", subcore_axis_name="s")
_SC_PARAMS = pltpu.CompilerParams(use_tc_tiling_on_sc=False)


def _counts_body(key_hbm, out_hbm, idxk, ones, zbuf, acc):
    cid = lax.axis_index("c")
    sid = lax.axis_index("s")
    wid = sid * 2 + cid

    def init_ones(i, _):
        ones[pl.ds(i * 16, 16)] = jnp.ones((16,), jnp.float32)
        return 0

    lax.fori_loop(0, C // 16, init_ones, 0)

    def init_z(i, _):
        zbuf[pl.ds(i * 16, 16)] = jnp.zeros((16,), jnp.float32)
        return 0

    lax.fori_loop(0, CNT_SLICE // 16, init_z, 0)
    pltpu.sync_copy(zbuf, acc.at[pl.ds(sid * CNT_SLICE, CNT_SLICE)])
    plsc.subcore_barrier()

    def superchunk(s, _):
        pltpu.sync_copy(key_hbm.at[wid * NSUPER + s], idxk)
        for b in range(SUPER):
            pltpu.sync_copy(ones, acc.at[idxk.at[b]], add=True)
        return 0

    lax.fori_loop(0, NSUPER, superchunk, 0)
    plsc.subcore_barrier()
    pltpu.sync_copy(acc.at[pl.ds(sid * CNT_SLICE, CNT_SLICE)],
                    out_hbm.at[cid, 0, pl.ds(sid * CNT_SLICE, CNT_SLICE)])


_sc_counts = functools.partial(
    pl.kernel,
    out_type=jax.ShapeDtypeStruct((2, 1, NKEY_PAD), jnp.float32),
    mesh=_MESH,
    compiler_params=_SC_PARAMS,
    scratch_types=[
        pltpu.VMEM((SUPER, C), jnp.int32),
        pltpu.VMEM((C,), jnp.float32),
        pltpu.VMEM((CNT_SLICE,), jnp.float32),
        pltpu.VMEM_SHARED((NKEY_PAD,), jnp.float32),
    ],
)(_counts_body)


def _make_edge_body(H, NBUF):
    def body(rowi_hbm, key_hbm, dst_hbm, tab_hbm, inv_hbm, out_hbm, *scr):
        idxr, idxk, idxd = scr[0:3]
        rows = scr[3:3 + NBUF]
        wv = scr[3 + NBUF:3 + 2 * NBUF]
        zbuf = scr[3 + 2 * NBUF]
        acc = scr[4 + 2 * NBUF]
        gsem = scr[5 + 2 * NBUF:5 + 3 * NBUF]
        wsem = scr[5 + 3 * NBUF:5 + 4 * NBUF]
        ssem = scr[5 + 4 * NBUF:5 + 5 * NBUF]
        cid = lax.axis_index("c")
        sid = lax.axis_index("s")
        wid = sid * 2 + cid

        def zrow(r, _):
            for c in range(H // 16):
                zbuf[r, pl.ds(c * 16, 16)] = jnp.zeros((16,), jnp.float32)
            return 0

        lax.fori_loop(0, ZBUF_ROWS, zrow, 0)

        def zcopy(z, _):
            pltpu.sync_copy(zbuf, acc.at[pl.ds(sid * ZROWS + z * ZBUF_ROWS,
                                               ZBUF_ROWS)])
            return 0

        lax.fori_loop(0, ZROWS // ZBUF_ROWS, zcopy, 0)

        @pl.when(sid == 0)
        def _zero_tail():
            def ztail(z, _):
                pltpu.sync_copy(
                    zbuf, acc.at[pl.ds(16 * ZROWS + z * ZBUF_ROWS,
                                       ZBUF_ROWS)])
                return 0

            lax.fori_loop(0, (ACC_ROWS - 16 * ZROWS) // ZBUF_ROWS, ztail, 0)

        plsc.subcore_barrier()

        def superchunk(s, _):
            srow = wid * NSUPER + s
            pltpu.sync_copy(rowi_hbm.at[srow], idxr)
            pltpu.sync_copy(key_hbm.at[srow], idxk)
            pltpu.sync_copy(dst_hbm.at[srow], idxd)
            gops = {}
            sops = {}

            def start_gather(b):
                p = b % NBUF
                gops[b] = (
                    pltpu.async_copy(tab_hbm.at[idxr.at[b]], rows[p],
                                     gsem[p]),
                    pltpu.async_copy(inv_hbm.at[idxk.at[b]], wv[p],
                                     wsem[p]),
                )

            start_gather(0)
            for b in range(SUPER):
                p = b % NBUF
                c1, c2 = gops.pop(b)
                c1.wait()
                c2.wait()
                if b + 1 < SUPER:
                    if b + 1 - NBUF >= 0:
                        sops.pop(b + 1 - NBUF).wait()
                    start_gather(b + 1)

                def scale(e, _, p=p):
                    w = wv[p][e]
                    for c in range(H // 16):
                        sl = pl.ds(c * 16, 16)
                        rows[p][e, sl] = rows[p][e, sl] * w
                    return 0

                lax.fori_loop(0, C, scale, 0)
                sops[b] = pltpu.async_copy(rows[p], acc.at[idxd.at[b]],
                                           ssem[p], add=True)
            for b in sorted(sops):
                sops.pop(b).wait()
            return 0

        lax.fori_loop(0, NSUPER, superchunk, 0)
        plsc.subcore_barrier()
        pltpu.sync_copy(acc.at[pl.ds(sid * ZROWS, ZROWS)],
                        out_hbm.at[cid, pl.ds(sid * ZROWS, ZROWS)])

        @pl.when(sid == 0)
        def _copy_tail():
            pltpu.sync_copy(acc.at[pl.ds(16 * ZROWS, N - 16 * ZROWS)],
                            out_hbm.at[cid, pl.ds(16 * ZROWS, N - 16 * ZROWS)])

    return body


def _sc_edge(H, NBUF):
    return functools.partial(
        pl.kernel,
        out_type=jax.ShapeDtypeStruct((2, N, H), jnp.float32),
        mesh=_MESH,
        compiler_params=_SC_PARAMS,
        scratch_types=(
            [pltpu.VMEM((SUPER, C), jnp.int32)] * 3
            + [pltpu.VMEM((C, H), jnp.float32)] * NBUF
            + [pltpu.VMEM((C, 16), jnp.float32)] * NBUF
            + [pltpu.VMEM((ZBUF_ROWS, H), jnp.float32)]
            + [pltpu.VMEM_SHARED((ACC_ROWS, H), jnp.float32)]
            + [pltpu.SemaphoreType.DMA] * (3 * NBUF)
        ),
    )(_make_edge_body(H, NBUF))


_sc_edge64 = _sc_edge(HID, 3)
_sc_edge128 = _sc_edge(D_OUT, 2)


def _inv_body(cnt_ref, inv_ref):
    c = cnt_ref[0] + cnt_ref[1]
    inv_ref[...] = jnp.where(c > 0.0, 1.0 / jnp.maximum(c, 1.0), 0.0)


def _tc_inv(cnt):
    return pl.pallas_call(
        _inv_body,
        out_shape=jax.ShapeDtypeStruct((NKEY_PAD,), jnp.float32),
    )(cnt)


def _mm1_body(x_ref, w_ref, b_ref, p_ref, t_ref):
    acc = jnp.dot(x_ref[...], w_ref[...],
                  preferred_element_type=jnp.float32) + b_ref[...]
    p_ref[...] = acc[:, :HID]
    t_ref[...] = acc[:, HID:]


def _tc_mm1(x, wcat, bcat):
    blk = 1000
    return pl.pallas_call(
        _mm1_body,
        grid=(N // blk,),
        in_specs=[
            pl.BlockSpec((blk, D_IN), lambda i: (i, 0)),
            pl.BlockSpec((D_IN, HID + R * HID), lambda i: (0, 0)),
            pl.BlockSpec((1, HID + R * HID), lambda i: (0, 0)),
        ],
        out_specs=[
            pl.BlockSpec((blk, HID), lambda i: (i, 0)),
            pl.BlockSpec((blk, R * HID), lambda i: (i, 0)),
        ],
        out_shape=[
            jax.ShapeDtypeStruct((N, HID), jnp.float32),
            jax.ShapeDtypeStruct((N, R * HID), jnp.float32),
        ],
    )(x, wcat, bcat)


def _mm2_body(p1_ref, a0_ref, a1_ref, w_ref, b_ref, p_ref, t_ref):
    h = jnp.maximum(p1_ref[...] + a0_ref[...] + a1_ref[...], 0.0)
    acc = jnp.dot(h, w_ref[...],
                  preferred_element_type=jnp.float32) + b_ref[...]
    p_ref[...] = acc[:, :D_OUT]
    t_ref[...] = acc[:, D_OUT:]


def _tc_mm2(p1, a0, a1, wcat, bcat):
    blk = 1000
    return pl.pallas_call(
        _mm2_body,
        grid=(N // blk,),
        in_specs=[
            pl.BlockSpec((blk, HID), lambda i: (i, 0)),
            pl.BlockSpec((blk, HID), lambda i: (i, 0)),
            pl.BlockSpec((blk, HID), lambda i: (i, 0)),
            pl.BlockSpec((HID, D_OUT + R * D_OUT), lambda i: (0, 0)),
            pl.BlockSpec((1, D_OUT + R * D_OUT), lambda i: (0, 0)),
        ],
        out_specs=[
            pl.BlockSpec((blk, D_OUT), lambda i: (i, 0)),
            pl.BlockSpec((blk, R * D_OUT), lambda i: (i, 0)),
        ],
        out_shape=[
            jax.ShapeDtypeStruct((N, D_OUT), jnp.float32),
            jax.ShapeDtypeStruct((N, R * D_OUT), jnp.float32),
        ],
    )(p1, a0, a1, wcat, bcat)


def _add3_body(p_ref, a0_ref, a1_ref, o_ref):
    o_ref[...] = p_ref[...] + a0_ref[...] + a1_ref[...]


def _tc_add3(p, a0, a1):
    blk = 1000
    return pl.pallas_call(
        _add3_body,
        grid=(N // blk,),
        in_specs=[pl.BlockSpec((blk, D_OUT), lambda i: (i, 0))] * 3,
        out_specs=pl.BlockSpec((blk, D_OUT), lambda i: (i, 0)),
        out_shape=jax.ShapeDtypeStruct((N, D_OUT), jnp.float32),
    )(p, a0, a1)


def kernel(x, edge_index, edge_type, W1, root1, b1, W2, root2, b2):
    src = edge_index[0].astype(jnp.int32)
    dst = edge_index[1].astype(jnp.int32)
    typ = edge_type.astype(jnp.int32)
    pad = E_PAD - E
    rowi = jnp.concatenate([src * R + typ, jnp.zeros((pad,), jnp.int32)])
    keyi = jnp.concatenate([dst * R + typ, jnp.full((pad,), NKEY, jnp.int32)])
    dsti = jnp.concatenate([dst, jnp.full((pad,), N, jnp.int32)])
    shape3 = (NW * NSUPER, SUPER, C)
    rowi = rowi.reshape(shape3)
    keyi = keyi.reshape(shape3)
    dsti = dsti.reshape(shape3)

    wcat1 = jnp.concatenate(
        [root1, jnp.transpose(W1, (1, 0, 2)).reshape(D_IN, R * HID)], axis=1)
    bcat1 = jnp.concatenate(
        [b1, jnp.zeros((R * HID,), jnp.float32)])[None, :]
    wcat2 = jnp.concatenate(
        [root2, jnp.transpose(W2, (1, 0, 2)).reshape(HID, R * D_OUT)], axis=1)
    bcat2 = jnp.concatenate(
        [b2, jnp.zeros((R * D_OUT,), jnp.float32)])[None, :]

    cnt = _sc_counts(keyi).reshape(2, NKEY_PAD)
    inv = _tc_inv(cnt)
    inv16 = jnp.broadcast_to(inv[:, None], (NKEY_PAD, 16))

    p1, t1 = _tc_mm1(x, wcat1, bcat1)
    a1 = _sc_edge64(rowi, keyi, dsti, t1.reshape(N * R, HID), inv16)
    p2, t2 = _tc_mm2(p1, a1[0], a1[1], wcat2, bcat2)
    a2 = _sc_edge128(rowi, keyi, dsti, t2.reshape(N * R, D_OUT), inv16)
    return _tc_add3(p2, a2[0], a2[1])


# speedup vs baseline: 15.7977x; 1.1366x over previous
import functools

import jax
import jax.numpy as jnp
from jax import lax
from jax.experimental import pallas as pl
from jax.experimental.pallas import tpu as pltpu
from jax.experimental.pallas import tpu_sc as plsc

N = 10000
E = 320000
D_IN = 128
HID = 64
D_OUT = 128
R = 8

NW = 32
E_PAD = 327680
EW = E_PAD // NW
C = 128
NCHUNK = EW // C
NSUPER = 8
SUPER = NCHUNK // NSUPER
SUP_FAST = 12
SUP_SLOW = 4
FAST_TOTAL = 16 * SUP_FAST
NKEY = N * R
NKEY_PAD = 81920
CNT_SLICE = NKEY_PAD // 16
ACC_ROWS = N + 16
ZROWS = 624
ZBUF_ROWS = 8

_MESH = plsc.VectorSubcoreMesh(core_axis_name="c", subcore_axis_name="s")
_SC_PARAMS = pltpu.CompilerParams(use_tc_tiling_on_sc=False)


def _counts_body(key_hbm, out_hbm, idxk, ones, zbuf, acc):
    cid = lax.axis_index("c")
    sid = lax.axis_index("s")
    wid = sid * 2 + cid

    def init_ones(i, _):
        ones[pl.ds(i * 16, 16)] = jnp.ones((16,), jnp.float32)
        return 0

    lax.fori_loop(0, C // 16, init_ones, 0)

    def init_z(i, _):
        zbuf[pl.ds(i * 16, 16)] = jnp.zeros((16,), jnp.float32)
        return 0

    lax.fori_loop(0, CNT_SLICE // 16, init_z, 0)
    pltpu.sync_copy(zbuf, acc.at[pl.ds(sid * CNT_SLICE, CNT_SLICE)])
    plsc.subcore_barrier()

    def superchunk(s, _):
        pltpu.sync_copy(key_hbm.at[wid * NSUPER + s], idxk)
        for b in range(SUPER):
            pltpu.sync_copy(ones, acc.at[idxk.at[b]], add=True)
        return 0

    lax.fori_loop(0, NSUPER, superchunk, 0)
    plsc.subcore_barrier()
    pltpu.sync_copy(acc.at[pl.ds(sid * CNT_SLICE, CNT_SLICE)],
                    out_hbm.at[cid, 0, pl.ds(sid * CNT_SLICE, CNT_SLICE)])


_sc_counts = functools.partial(
    pl.kernel,
    out_type=jax.ShapeDtypeStruct((2, 1, NKEY_PAD), jnp.float32),
    mesh=_MESH,
    compiler_params=_SC_PARAMS,
    scratch_types=[
        pltpu.VMEM((SUPER, C), jnp.int32),
        pltpu.VMEM((C,), jnp.float32),
        pltpu.VMEM((CNT_SLICE,), jnp.float32),
        pltpu.VMEM_SHARED((NKEY_PAD,), jnp.float32),
    ],
)(_counts_body)


def _make_edge_body(H, NBUF):
    def body(rowi_hbm, key_hbm, dst_hbm, tab_hbm, inv_hbm, out_hbm, *scr):
        idxr, idxk, idxd = scr[0:3]
        rows = scr[3:3 + NBUF]
        wv = scr[3 + NBUF:3 + 2 * NBUF]
        zbuf = scr[3 + 2 * NBUF]
        acc = scr[4 + 2 * NBUF]
        gsem = scr[5 + 2 * NBUF:5 + 3 * NBUF]
        wsem = scr[5 + 3 * NBUF:5 + 4 * NBUF]
        ssem = scr[5 + 4 * NBUF:5 + 5 * NBUF]
        cid = lax.axis_index("c")
        sid = lax.axis_index("s")
        wid = sid * 2 + cid

        def zrow(r, _):
            for c in range(H // 16):
                zbuf[r, pl.ds(c * 16, 16)] = jnp.zeros((16,), jnp.float32)
            return 0

        lax.fori_loop(0, ZBUF_ROWS, zrow, 0)

        def zcopy(z, _):
            pltpu.sync_copy(zbuf, acc.at[pl.ds(sid * ZROWS + z * ZBUF_ROWS,
                                               ZBUF_ROWS)])
            return 0

        lax.fori_loop(0, ZROWS // ZBUF_ROWS, zcopy, 0)

        @pl.when(sid == 0)
        def _zero_tail():
            def ztail(z, _):
                pltpu.sync_copy(
                    zbuf, acc.at[pl.ds(16 * ZROWS + z * ZBUF_ROWS,
                                       ZBUF_ROWS)])
                return 0

            lax.fori_loop(0, (ACC_ROWS - 16 * ZROWS) // ZBUF_ROWS, ztail, 0)

        plsc.subcore_barrier()

        nsup = jnp.where(cid == 0, SUP_FAST, SUP_SLOW)
        sbase = jnp.where(cid == 0, sid * SUP_FAST,
                          FAST_TOTAL + sid * SUP_SLOW)

        def superchunk(s, _):
            srow = sbase + s
            pltpu.sync_copy(rowi_hbm.at[srow], idxr)
            pltpu.sync_copy(key_hbm.at[srow], idxk)
            pltpu.sync_copy(dst_hbm.at[srow], idxd)
            gops = {}
            sops = {}

            def start_gather(b):
                p = b % NBUF
                gops[b] = (
                    pltpu.async_copy(tab_hbm.at[idxr.at[b]], rows[p],
                                     gsem[p]),
                    pltpu.async_copy(inv_hbm.at[idxk.at[b]], wv[p],
                                     wsem[p]),
                )

            start_gather(0)
            for b in range(SUPER):
                p = b % NBUF
                c1, c2 = gops.pop(b)
                c1.wait()
                c2.wait()
                if b + 1 < SUPER:
                    if b + 1 - NBUF >= 0:
                        sops.pop(b + 1 - NBUF).wait()
                    start_gather(b + 1)

                def scale(e, _, p=p):
                    w = wv[p][e]
                    for c in range(H // 16):
                        sl = pl.ds(c * 16, 16)
                        rows[p][e, sl] = rows[p][e, sl] * w
                    return 0

                lax.fori_loop(0, C, scale, 0)
                sops[b] = pltpu.async_copy(rows[p], acc.at[idxd.at[b]],
                                           ssem[p], add=True)
            for b in sorted(sops):
                sops.pop(b).wait()
            return 0

        lax.fori_loop(0, nsup, superchunk, 0)
        plsc.subcore_barrier()
        pltpu.sync_copy(acc.at[pl.ds(sid * ZROWS, ZROWS)],
                        out_hbm.at[cid, pl.ds(sid * ZROWS, ZROWS)])

        @pl.when(sid == 0)
        def _copy_tail():
            pltpu.sync_copy(acc.at[pl.ds(16 * ZROWS, N - 16 * ZROWS)],
                            out_hbm.at[cid, pl.ds(16 * ZROWS, N - 16 * ZROWS)])

    return body


def _sc_edge(H, NBUF):
    return functools.partial(
        pl.kernel,
        out_type=jax.ShapeDtypeStruct((2, N, H), jnp.float32),
        mesh=_MESH,
        compiler_params=_SC_PARAMS,
        scratch_types=(
            [pltpu.VMEM((SUPER, C), jnp.int32)] * 3
            + [pltpu.VMEM((C, H), jnp.float32)] * NBUF
            + [pltpu.VMEM((C, 16), jnp.float32)] * NBUF
            + [pltpu.VMEM((ZBUF_ROWS, H), jnp.float32)]
            + [pltpu.VMEM_SHARED((ACC_ROWS, H), jnp.float32)]
            + [pltpu.SemaphoreType.DMA] * (3 * NBUF)
        ),
    )(_make_edge_body(H, NBUF))


_sc_edge64 = _sc_edge(HID, 3)
_sc_edge128 = _sc_edge(D_OUT, 2)


def _inv_body(cnt_ref, inv_ref):
    c = cnt_ref[0] + cnt_ref[1]
    inv_ref[...] = jnp.where(c > 0.0, 1.0 / jnp.maximum(c, 1.0), 0.0)


def _tc_inv(cnt):
    return pl.pallas_call(
        _inv_body,
        out_shape=jax.ShapeDtypeStruct((NKEY_PAD,), jnp.float32),
    )(cnt)


def _mm1_body(x_ref, w_ref, b_ref, p_ref, t_ref):
    acc = jnp.dot(x_ref[...], w_ref[...],
                  preferred_element_type=jnp.float32) + b_ref[...]
    p_ref[...] = acc[:, :HID]
    t_ref[...] = acc[:, HID:]


def _tc_mm1(x, wcat, bcat):
    blk = 1000
    return pl.pallas_call(
        _mm1_body,
        grid=(N // blk,),
        in_specs=[
            pl.BlockSpec((blk, D_IN), lambda i: (i, 0)),
            pl.BlockSpec((D_IN, HID + R * HID), lambda i: (0, 0)),
            pl.BlockSpec((1, HID + R * HID), lambda i: (0, 0)),
        ],
        out_specs=[
            pl.BlockSpec((blk, HID), lambda i: (i, 0)),
            pl.BlockSpec((blk, R * HID), lambda i: (i, 0)),
        ],
        out_shape=[
            jax.ShapeDtypeStruct((N, HID), jnp.float32),
            jax.ShapeDtypeStruct((N, R * HID), jnp.float32),
        ],
    )(x, wcat, bcat)


def _mm2_body(p1_ref, a0_ref, a1_ref, w_ref, b_ref, p_ref, t_ref):
    h = jnp.maximum(p1_ref[...] + a0_ref[...] + a1_ref[...], 0.0)
    acc = jnp.dot(h, w_ref[...],
                  preferred_element_type=jnp.float32) + b_ref[...]
    p_ref[...] = acc[:, :D_OUT]
    t_ref[...] = acc[:, D_OUT:]


def _tc_mm2(p1, a0, a1, wcat, bcat):
    blk = 1000
    return pl.pallas_call(
        _mm2_body,
        grid=(N // blk,),
        in_specs=[
            pl.BlockSpec((blk, HID), lambda i: (i, 0)),
            pl.BlockSpec((blk, HID), lambda i: (i, 0)),
            pl.BlockSpec((blk, HID), lambda i: (i, 0)),
            pl.BlockSpec((HID, D_OUT + R * D_OUT), lambda i: (0, 0)),
            pl.BlockSpec((1, D_OUT + R * D_OUT), lambda i: (0, 0)),
        ],
        out_specs=[
            pl.BlockSpec((blk, D_OUT), lambda i: (i, 0)),
            pl.BlockSpec((blk, R * D_OUT), lambda i: (i, 0)),
        ],
        out_shape=[
            jax.ShapeDtypeStruct((N, D_OUT), jnp.float32),
            jax.ShapeDtypeStruct((N, R * D_OUT), jnp.float32),
        ],
    )(p1, a0, a1, wcat, bcat)


def _add3_body(p_ref, a0_ref, a1_ref, o_ref):
    o_ref[...] = p_ref[...] + a0_ref[...] + a1_ref[...]


def _tc_add3(p, a0, a1):
    blk = 1000
    return pl.pallas_call(
        _add3_body,
        grid=(N // blk,),
        in_specs=[pl.BlockSpec((blk, D_OUT), lambda i: (i, 0))] * 3,
        out_specs=pl.BlockSpec((blk, D_OUT), lambda i: (i, 0)),
        out_shape=jax.ShapeDtypeStruct((N, D_OUT), jnp.float32),
    )(p, a0, a1)


def kernel(x, edge_index, edge_type, W1, root1, b1, W2, root2, b2):
    src = edge_index[0].astype(jnp.int32)
    dst = edge_index[1].astype(jnp.int32)
    typ = edge_type.astype(jnp.int32)
    pad = E_PAD - E
    rowi = jnp.concatenate([src * R + typ, jnp.zeros((pad,), jnp.int32)])
    keyi = jnp.concatenate([dst * R + typ, jnp.full((pad,), NKEY, jnp.int32)])
    dsti = jnp.concatenate([dst, jnp.full((pad,), N, jnp.int32)])
    shape3 = (NW * NSUPER, SUPER, C)
    rowi = rowi.reshape(shape3)
    keyi = keyi.reshape(shape3)
    dsti = dsti.reshape(shape3)

    wcat1 = jnp.concatenate(
        [root1, jnp.transpose(W1, (1, 0, 2)).reshape(D_IN, R * HID)], axis=1)
    bcat1 = jnp.concatenate(
        [b1, jnp.zeros((R * HID,), jnp.float32)])[None, :]
    wcat2 = jnp.concatenate(
        [root2, jnp.transpose(W2, (1, 0, 2)).reshape(HID, R * D_OUT)], axis=1)
    bcat2 = jnp.concatenate(
        [b2, jnp.zeros((R * D_OUT,), jnp.float32)])[None, :]

    cnt = _sc_counts(keyi).reshape(2, NKEY_PAD)
    inv = _tc_inv(cnt)
    inv16 = jnp.broadcast_to(inv[:, None], (NKEY_PAD, 16))

    p1, t1 = _tc_mm1(x, wcat1, bcat1)
    a1 = _sc_edge64(rowi, keyi, dsti, t1.reshape(N * R, HID), inv16)
    p2, t2 = _tc_mm2(p1, a1[0], a1[1], wcat2, bcat2)
    a2 = _sc_edge128(rowi, keyi, dsti, t2.reshape(N * R, D_OUT), inv16)
    return _tc_add3(p2, a2[0], a2[1])


# speedup vs baseline: 21.0083x; 1.3298x over previous
import functools

import jax
import jax.numpy as jnp
from jax import lax
from jax.experimental import pallas as pl
from jax.experimental.pallas import tpu as pltpu
from jax.experimental.pallas import tpu_sc as plsc

N = 10000
E = 320000
D_IN = 128
HID = 64
D_OUT = 128
R = 8

NW = 32
E_PAD = 327680
EW = E_PAD // NW
C = 128
NCHUNK = EW // C
NSUPER = 8
SUPER = NCHUNK // NSUPER
SUP_FAST = 12
SUP_SLOW = 4
FAST_TOTAL = 16 * SUP_FAST
NKEY = N * R
NKEY_PAD = 81920
CNT_SLICE = NKEY_PAD // 16
ACC_ROWS = N + 16
ZROWS = 624
ZBUF_ROWS = 8

_MESH = plsc.VectorSubcoreMesh(core_axis_name="c", subcore_axis_name="s")
_SC_PARAMS = pltpu.CompilerParams(use_tc_tiling_on_sc=False,
                                  needs_layout_passes=False)


def _counts_body(key_hbm, out_hbm, idxk, ones, zbuf, acc):
    cid = lax.axis_index("c")
    sid = lax.axis_index("s")
    wid = sid * 2 + cid

    def init_ones(i, _):
        ones[pl.ds(i * 16, 16)] = jnp.ones((16,), jnp.float32)
        return 0

    lax.fori_loop(0, C // 16, init_ones, 0)

    def init_z(i, _):
        zbuf[pl.ds(i * 16, 16)] = jnp.zeros((16,), jnp.float32)
        return 0

    lax.fori_loop(0, CNT_SLICE // 16, init_z, 0)
    pltpu.sync_copy(zbuf, acc.at[pl.ds(sid * CNT_SLICE, CNT_SLICE)])
    plsc.subcore_barrier()

    def superchunk(s, _):
        pltpu.sync_copy(key_hbm.at[wid * NSUPER + s], idxk)
        for b in range(SUPER):
            pltpu.sync_copy(ones, acc.at[idxk.at[b]], add=True)
        return 0

    lax.fori_loop(0, NSUPER, superchunk, 0)
    plsc.subcore_barrier()
    pltpu.sync_copy(acc.at[pl.ds(sid * CNT_SLICE, CNT_SLICE)],
                    out_hbm.at[cid, 0, pl.ds(sid * CNT_SLICE, CNT_SLICE)])


_sc_counts = functools.partial(
    pl.kernel,
    out_type=jax.ShapeDtypeStruct((2, 1, NKEY_PAD), jnp.float32),
    mesh=_MESH,
    compiler_params=_SC_PARAMS,
    scratch_types=[
        pltpu.VMEM((SUPER, C), jnp.int32),
        pltpu.VMEM((C,), jnp.float32),
        pltpu.VMEM((CNT_SLICE,), jnp.float32),
        pltpu.VMEM_SHARED((NKEY_PAD,), jnp.float32),
    ],
)(_counts_body)


def _make_edge_body(H, NBUF):
    def body(rowi_hbm, key_hbm, dst_hbm, tab_hbm, inv_hbm, out_hbm, *scr):
        idxr, idxk, idxd = scr[0:3]
        rows = scr[3:3 + NBUF]
        wv = scr[3 + NBUF:3 + 2 * NBUF]
        zbuf = scr[3 + 2 * NBUF]
        acc = scr[4 + 2 * NBUF]
        gsem = scr[5 + 2 * NBUF:5 + 3 * NBUF]
        wsem = scr[5 + 3 * NBUF:5 + 4 * NBUF]
        ssem = scr[5 + 4 * NBUF:5 + 5 * NBUF]
        cid = lax.axis_index("c")
        sid = lax.axis_index("s")
        wid = sid * 2 + cid

        def zrow(r, _):
            for c in range(H // 32):
                zbuf[r, pl.ds(c * 32, 32)] = jnp.zeros((32,), jnp.bfloat16)
            return 0

        lax.fori_loop(0, ZBUF_ROWS, zrow, 0)

        def zcopy(z, _):
            pltpu.sync_copy(zbuf, acc.at[pl.ds(sid * ZROWS + z * ZBUF_ROWS,
                                               ZBUF_ROWS)])
            return 0

        lax.fori_loop(0, ZROWS // ZBUF_ROWS, zcopy, 0)

        @pl.when(sid == 0)
        def _zero_tail():
            def ztail(z, _):
                pltpu.sync_copy(
                    zbuf, acc.at[pl.ds(16 * ZROWS + z * ZBUF_ROWS,
                                       ZBUF_ROWS)])
                return 0

            lax.fori_loop(0, (ACC_ROWS - 16 * ZROWS) // ZBUF_ROWS, ztail, 0)

        plsc.subcore_barrier()

        nsup = jnp.where(cid == 0, SUP_FAST, SUP_SLOW)
        sbase = jnp.where(cid == 0, sid * SUP_FAST,
                          FAST_TOTAL + sid * SUP_SLOW)

        def superchunk(s, _):
            srow = sbase + s
            pltpu.sync_copy(rowi_hbm.at[srow], idxr)
            pltpu.sync_copy(key_hbm.at[srow], idxk)
            pltpu.sync_copy(dst_hbm.at[srow], idxd)
            gops = {}
            sops = {}

            def start_gather(b):
                p = b % NBUF
                gops[b] = (
                    pltpu.async_copy(tab_hbm.at[idxr.at[b]], rows[p],
                                     gsem[p]),
                    pltpu.async_copy(inv_hbm.at[idxk.at[b]], wv[p],
                                     wsem[p]),
                )

            start_gather(0)
            for b in range(SUPER):
                p = b % NBUF
                c1, c2 = gops.pop(b)
                c1.wait()
                c2.wait()
                if b + 1 < SUPER:
                    if b + 1 - NBUF >= 0:
                        sops.pop(b + 1 - NBUF).wait()
                    start_gather(b + 1)

                def scale(e, _, p=p):
                    w = plsc.pack(wv[p][e], wv[p][e],
                                  format=plsc.PackFormat.INTERLEAVED)
                    for c in range(H // 32):
                        sl = pl.ds(c * 32, 32)
                        rows[p][e, sl] = rows[p][e, sl] * w
                    return 0

                lax.fori_loop(0, C, scale, 0)
                sops[b] = pltpu.async_copy(rows[p], acc.at[idxd.at[b]],
                                           ssem[p], add=True)
            for b in sorted(sops):
                sops.pop(b).wait()
            return 0

        lax.fori_loop(0, nsup, superchunk, 0)
        plsc.subcore_barrier()
        pltpu.sync_copy(acc.at[pl.ds(sid * ZROWS, ZROWS)],
                        out_hbm.at[cid, pl.ds(sid * ZROWS, ZROWS)])

        @pl.when(sid == 0)
        def _copy_tail():
            pltpu.sync_copy(acc.at[pl.ds(16 * ZROWS, N - 16 * ZROWS)],
                            out_hbm.at[cid, pl.ds(16 * ZROWS, N - 16 * ZROWS)])

    return body


def _sc_edge(H, NBUF):
    return functools.partial(
        pl.kernel,
        out_type=jax.ShapeDtypeStruct((2, N, H), jnp.bfloat16),
        mesh=_MESH,
        compiler_params=_SC_PARAMS,
        scratch_types=(
            [pltpu.VMEM((SUPER, C), jnp.int32)] * 3
            + [pltpu.VMEM((C, H), jnp.bfloat16)] * NBUF
            + [pltpu.VMEM((C, 16), jnp.float32)] * NBUF
            + [pltpu.VMEM((ZBUF_ROWS, H), jnp.bfloat16)]
            + [pltpu.VMEM_SHARED((ACC_ROWS, H), jnp.bfloat16)]
            + [pltpu.SemaphoreType.DMA] * (3 * NBUF)
        ),
    )(_make_edge_body(H, NBUF))


_sc_edge64 = _sc_edge(HID, 3)
_sc_edge128 = _sc_edge(D_OUT, 2)


def _inv_body(cnt_ref, inv_ref):
    c = cnt_ref[0] + cnt_ref[1]
    inv_ref[...] = jnp.where(c > 0.0, 1.0 / jnp.maximum(c, 1.0), 0.0)


def _tc_inv(cnt):
    return pl.pallas_call(
        _inv_body,
        out_shape=jax.ShapeDtypeStruct((NKEY_PAD,), jnp.float32),
    )(cnt)


def _mm1_body(x_ref, w_ref, b_ref, p_ref, t_ref):
    acc = jnp.dot(x_ref[...], w_ref[...],
                  preferred_element_type=jnp.float32) + b_ref[...]
    p_ref[...] = acc[:, :HID]
    t_ref[...] = acc[:, HID:].astype(jnp.bfloat16)


def _tc_mm1(x, wcat, bcat):
    blk = 2000
    return pl.pallas_call(
        _mm1_body,
        grid=(N // blk,),
        in_specs=[
            pl.BlockSpec((blk, D_IN), lambda i: (i, 0)),
            pl.BlockSpec((D_IN, HID + R * HID), lambda i: (0, 0)),
            pl.BlockSpec((1, HID + R * HID), lambda i: (0, 0)),
        ],
        out_specs=[
            pl.BlockSpec((blk, HID), lambda i: (i, 0)),
            pl.BlockSpec((blk, R * HID), lambda i: (i, 0)),
        ],
        out_shape=[
            jax.ShapeDtypeStruct((N, HID), jnp.float32),
            jax.ShapeDtypeStruct((N, R * HID), jnp.bfloat16),
        ],
    )(x, wcat, bcat)


def _mm2_body(p1_ref, a0_ref, a1_ref, w_ref, b_ref, p_ref, t_ref):
    h = jnp.maximum(p1_ref[...] + a0_ref[...].astype(jnp.float32)
                    + a1_ref[...].astype(jnp.float32), 0.0)
    acc = jnp.dot(h, w_ref[...],
                  preferred_element_type=jnp.float32) + b_ref[...]
    p_ref[...] = acc[:, :D_OUT]
    t_ref[...] = acc[:, D_OUT:].astype(jnp.bfloat16)


def _tc_mm2(p1, a0, a1, wcat, bcat):
    blk = 2000
    return pl.pallas_call(
        _mm2_body,
        grid=(N // blk,),
        in_specs=[
            pl.BlockSpec((blk, HID), lambda i: (i, 0)),
            pl.BlockSpec((blk, HID), lambda i: (i, 0)),
            pl.BlockSpec((blk, HID), lambda i: (i, 0)),
            pl.BlockSpec((HID, D_OUT + R * D_OUT), lambda i: (0, 0)),
            pl.BlockSpec((1, D_OUT + R * D_OUT), lambda i: (0, 0)),
        ],
        out_specs=[
            pl.BlockSpec((blk, D_OUT), lambda i: (i, 0)),
            pl.BlockSpec((blk, R * D_OUT), lambda i: (i, 0)),
        ],
        out_shape=[
            jax.ShapeDtypeStruct((N, D_OUT), jnp.float32),
            jax.ShapeDtypeStruct((N, R * D_OUT), jnp.bfloat16),
        ],
    )(p1, a0, a1, wcat, bcat)


def _add3_body(p_ref, a0_ref, a1_ref, o_ref):
    o_ref[...] = (p_ref[...] + a0_ref[...].astype(jnp.float32)
                  + a1_ref[...].astype(jnp.float32))


def _tc_add3(p, a0, a1):
    blk = 2000
    return pl.pallas_call(
        _add3_body,
        grid=(N // blk,),
        in_specs=[pl.BlockSpec((blk, D_OUT), lambda i: (i, 0))] * 3,
        out_specs=pl.BlockSpec((blk, D_OUT), lambda i: (i, 0)),
        out_shape=jax.ShapeDtypeStruct((N, D_OUT), jnp.float32),
    )(p, a0, a1)


def kernel(x, edge_index, edge_type, W1, root1, b1, W2, root2, b2):
    src = edge_index[0].astype(jnp.int32)
    dst = edge_index[1].astype(jnp.int32)
    typ = edge_type.astype(jnp.int32)
    pad = E_PAD - E
    rowi = jnp.concatenate([src * R + typ, jnp.zeros((pad,), jnp.int32)])
    keyi = jnp.concatenate([dst * R + typ, jnp.full((pad,), NKEY, jnp.int32)])
    dsti = jnp.concatenate([dst, jnp.full((pad,), N, jnp.int32)])
    shape3 = (NW * NSUPER, SUPER, C)
    rowi = rowi.reshape(shape3)
    keyi = keyi.reshape(shape3)
    dsti = dsti.reshape(shape3)

    wcat1 = jnp.concatenate(
        [root1, jnp.transpose(W1, (1, 0, 2)).reshape(D_IN, R * HID)], axis=1)
    bcat1 = jnp.concatenate(
        [b1, jnp.zeros((R * HID,), jnp.float32)])[None, :]
    wcat2 = jnp.concatenate(
        [root2, jnp.transpose(W2, (1, 0, 2)).reshape(HID, R * D_OUT)], axis=1)
    bcat2 = jnp.concatenate(
        [b2, jnp.zeros((R * D_OUT,), jnp.float32)])[None, :]

    cnt = _sc_counts(keyi).reshape(2, NKEY_PAD)
    inv = _tc_inv(cnt)
    inv16 = jnp.broadcast_to(inv[:, None], (NKEY_PAD, 16))

    p1, t1 = _tc_mm1(x, wcat1, bcat1)
    a1 = _sc_edge64(rowi, keyi, dsti, t1.reshape(N * R, HID), inv16)
    p2, t2 = _tc_mm2(p1, a1[0], a1[1], wcat2, bcat2)
    a2 = _sc_edge128(rowi, keyi, dsti, t2.reshape(N * R, D_OUT), inv16)
    return _tc_add3(p2, a2[0], a2[1])
